# Initial kernel scaffold; baseline (speedup 1.0000x reference)
import jax
import jax.numpy as jnp
from jax.experimental import pallas as pl


def kernel(x, edge_index, defense_indices, nop_index, gin_w1_0, gin_b1_0, gin_w2_0, gin_b2_0, gin_w1_1, gin_b1_1, gin_w2_1, gin_b2_1, gin_w1_2, gin_b1_2, gin_w2_2, gin_b2_2, policy_w, policy_b, value_w, value_b):
    raise NotImplementedError("write your pallas kernel here")


# speedup vs baseline: 4.6826x; 4.6826x over previous
import functools

import jax
import jax.numpy as jnp
from jax import lax
from jax.experimental import pallas as pl
from jax.experimental.pallas import tpu as pltpu
from jax.experimental.pallas import tpu_sc as plsc

N = 50000
E = 800000
H = 64
K = 64

NC = 2
NS = 16
HALF = N // NC
ACC_ROWS = 25600
TRASH_MASK = 511
EPT = 51200
EPAD = EPT * NS
CHUNK = 256
SUB = 128
NSUB = CHUNK // SUB
CHUNKS = EPT // CHUNK
ZROWS = 1600
OUT_PT = 1560
OUT_PT1 = 1560

@functools.lru_cache(maxsize=None)
def _mesh():
    return plsc.VectorSubcoreMesh(core_axis_name="c", subcore_axis_name="s",
                                  num_cores=NC, num_subcores=NS)


def _seg_sum_2d(h, src2, dst2):
    @functools.partial(
        pl.kernel,
        out_type=jax.ShapeDtypeStruct((N, H), jnp.float32),
        mesh=_mesh(),
        compiler_params=pltpu.CompilerParams(use_tc_tiling_on_sc=False),
        scratch_types=[
            pltpu.VMEM((NSUB, SUB), jnp.int32),
            pltpu.VMEM((NSUB, SUB), jnp.int32),
            pltpu.VMEM((NSUB, SUB), jnp.int32),
            pltpu.VMEM((CHUNK, H), jnp.float32),
            pltpu.VMEM_SHARED((ACC_ROWS, H), jnp.float32),
            pltpu.SemaphoreType.DMA,
            pltpu.SemaphoreType.DMA,
        ],
    )
    def k(h_hbm, src_hbm, dst_hbm, out_hbm, srcv, dstv, lidx, rows, acc,
          gsem, ssem):
        c = lax.axis_index("c")
        s = lax.axis_index("s")
        zero16 = jnp.zeros((16,), jnp.float32)
        iota = lax.broadcasted_iota(jnp.int32, (16,), 0)

        def zrow(i, _):
            for kk in range(H // 16):
                rows[i, pl.ds(kk * 16, 16)] = zero16
            return 0

        lax.fori_loop(0, 128, zrow, 0)

        def zacc(i, _):
            pltpu.sync_copy(rows.at[pl.ds(0, 128), :],
                            acc.at[pl.ds(s * ZROWS + i * 128, 128), :])
            return 0

        lax.fori_loop(0, 12, zacc, 0)
        pltpu.sync_copy(rows.at[pl.ds(0, 64), :],
                        acc.at[pl.ds(s * ZROWS + 1536, 64), :])
        plsc.subcore_barrier()

        base_half = c * HALF
        row0 = s * (CHUNKS * NSUB)

        def chunk(i, _):
            r0 = row0 + i * NSUB
            pltpu.sync_copy(src_hbm.at[pl.ds(r0, NSUB), :], srcv)
            pltpu.sync_copy(dst_hbm.at[pl.ds(r0, NSUB), :], dstv)
            gcps = [
                pltpu.async_copy(h_hbm.at[srcv.at[j]],
                                 rows.at[pl.ds(j * SUB, SUB), :], gsem)
                for j in range(NSUB)
            ]
            for j in range(NSUB):
                for kk in range(SUB // 16):
                    d = dstv[j, pl.ds(kk * 16, 16)]
                    lo = d - base_half
                    ok = (lo >= 0) & (lo < HALF)
                    tr = HALF + ((iota + (j * SUB + kk * 16) + s * 64 + i * 32)
                                 & TRASH_MASK)
                    lidx[j, pl.ds(kk * 16, 16)] = jnp.where(ok, lo, tr)
            for cp in gcps:
                cp.wait()
            scps = [
                pltpu.async_copy(rows.at[pl.ds(j * SUB, SUB), :],
                                 acc.at[lidx.at[j]], ssem, add=True)
                for j in range(NSUB)
            ]
            for cp in scps:
                cp.wait()
            return 0

        lax.fori_loop(0, CHUNKS, chunk, 0)
        plsc.subcore_barrier()

        def cout(i, _):
            pltpu.sync_copy(acc.at[pl.ds(s * OUT_PT + i * 128, 128), :],
                            rows.at[pl.ds(0, 128), :])
            pltpu.sync_copy(
                rows.at[pl.ds(0, 128), :],
                out_hbm.at[pl.ds(c * HALF + s * OUT_PT + i * 128, 128), :])
            return 0

        lax.fori_loop(0, OUT_PT // 128, cout, 0)
        t0 = (OUT_PT // 128) * 128
        tl = OUT_PT - t0
        pltpu.sync_copy(acc.at[pl.ds(s * OUT_PT + t0, tl), :],
                        rows.at[pl.ds(0, tl), :])
        pltpu.sync_copy(rows.at[pl.ds(0, tl), :],
                        out_hbm.at[pl.ds(c * HALF + s * OUT_PT + t0, tl), :])

        @pl.when(s == 0)
        def _tail():
            rem = HALF - NS * OUT_PT
            pltpu.sync_copy(acc.at[pl.ds(NS * OUT_PT, rem), :],
                            rows.at[pl.ds(0, rem), :])
            pltpu.sync_copy(rows.at[pl.ds(0, rem), :],
                            out_hbm.at[pl.ds(c * HALF + NS * OUT_PT, rem), :])

    return k(h, src2, dst2)


def _seg_sum_1d(x1, src2, dst2):
    @functools.partial(
        pl.kernel,
        out_type=jax.ShapeDtypeStruct((N,), jnp.float32),
        mesh=_mesh(),
        compiler_params=pltpu.CompilerParams(use_tc_tiling_on_sc=False),
        scratch_types=[
            pltpu.VMEM((NSUB, SUB), jnp.int32),
            pltpu.VMEM((NSUB, SUB), jnp.int32),
            pltpu.VMEM((NSUB, SUB), jnp.int32),
            pltpu.VMEM((CHUNK,), jnp.float32),
            pltpu.VMEM((128,), jnp.float32),
            pltpu.VMEM_SHARED((ACC_ROWS,), jnp.float32),
            pltpu.SemaphoreType.DMA,
            pltpu.SemaphoreType.DMA,
        ],
    )
    def k(x_hbm, src_hbm, dst_hbm, out_hbm, srcv, dstv, lidx, rows, zbuf, acc,
          gsem, ssem):
        c = lax.axis_index("c")
        s = lax.axis_index("s")
        zero16 = jnp.zeros((16,), jnp.float32)
        iota = lax.broadcasted_iota(jnp.int32, (16,), 0)

        for kk in range(8):
            zbuf[pl.ds(kk * 16, 16)] = zero16

        def zacc(i, _):
            pltpu.sync_copy(zbuf, acc.at[pl.ds(s * ZROWS + i * 128, 128)])
            return 0

        lax.fori_loop(0, 12, zacc, 0)
        pltpu.sync_copy(zbuf.at[pl.ds(0, 64)],
                        acc.at[pl.ds(s * ZROWS + 1536, 64)])
        plsc.subcore_barrier()

        base_half = c * HALF
        row0 = s * (CHUNKS * NSUB)

        def chunk(i, _):
            r0 = row0 + i * NSUB
            pltpu.sync_copy(src_hbm.at[pl.ds(r0, NSUB), :], srcv)
            pltpu.sync_copy(dst_hbm.at[pl.ds(r0, NSUB), :], dstv)
            gcps = [
                pltpu.async_copy(x_hbm.at[srcv.at[j]],
                                 rows.at[pl.ds(j * SUB, SUB)], gsem)
                for j in range(NSUB)
            ]
            for j in range(NSUB):
                for kk in range(SUB // 16):
                    d = dstv[j, pl.ds(kk * 16, 16)]
                    lo = d - base_half
                    ok = (lo >= 0) & (lo < HALF)
                    tr = HALF + ((iota + (j * SUB + kk * 16) + s * 64 + i * 32)
                                 & TRASH_MASK)
                    lidx[j, pl.ds(kk * 16, 16)] = jnp.where(ok, lo, tr)
            for cp in gcps:
                cp.wait()
            scps = [
                pltpu.async_copy(rows.at[pl.ds(j * SUB, SUB)],
                                 acc.at[lidx.at[j]], ssem, add=True)
                for j in range(NSUB)
            ]
            for cp in scps:
                cp.wait()
            return 0

        lax.fori_loop(0, CHUNKS, chunk, 0)
        plsc.subcore_barrier()

        def cout(i, _):
            pltpu.sync_copy(acc.at[pl.ds(s * OUT_PT1 + i * 128, 128)], zbuf)
            pltpu.sync_copy(
                zbuf, out_hbm.at[pl.ds(c * HALF + s * OUT_PT1 + i * 128, 128)])
            return 0

        lax.fori_loop(0, OUT_PT1 // 128, cout, 0)
        t0 = (OUT_PT1 // 128) * 128
        tl = OUT_PT1 - t0
        pltpu.sync_copy(acc.at[pl.ds(s * OUT_PT1 + t0, tl)],
                        zbuf.at[pl.ds(0, tl)])
        pltpu.sync_copy(zbuf.at[pl.ds(0, tl)],
                        out_hbm.at[pl.ds(c * HALF + s * OUT_PT1 + t0, tl)])

        @pl.when(s == 0)
        def _tail():
            rem = HALF - NS * OUT_PT1
            pltpu.sync_copy(acc.at[pl.ds(NS * OUT_PT1, rem)],
                            zbuf.at[pl.ds(0, rem)])
            pltpu.sync_copy(zbuf.at[pl.ds(0, rem)],
                            out_hbm.at[pl.ds(c * HALF + NS * OUT_PT1, rem)])

    return k(x1, src2, dst2)


_BLK = 2000
_GRID = N // _BLK


def _mlp0(x, agg0, w1, b1, w2, b2):
    def body(x_ref, a_ref, w1_ref, b1_ref, w2_ref, b2_ref, o_ref):
        z = x_ref[...] + a_ref[...]
        y = jnp.maximum(z * w1_ref[...] + b1_ref[...], 0.0)
        h = jnp.dot(y, w2_ref[...], preferred_element_type=jnp.float32)
        o_ref[...] = jnp.maximum(h + b2_ref[...], 0.0)

    return pl.pallas_call(
        body,
        grid=(_GRID,),
        in_specs=[
            pl.BlockSpec((_BLK, 1), lambda i: (i, 0)),
            pl.BlockSpec((_BLK, 1), lambda i: (i, 0)),
            pl.BlockSpec((1, H), lambda i: (0, 0)),
            pl.BlockSpec((1, H), lambda i: (0, 0)),
            pl.BlockSpec((H, H), lambda i: (0, 0)),
            pl.BlockSpec((1, H), lambda i: (0, 0)),
        ],
        out_specs=pl.BlockSpec((_BLK, H), lambda i: (i, 0)),
        out_shape=jax.ShapeDtypeStruct((N, H), jnp.float32),
    )(x, agg0, w1, b1, w2, b2)


def _mlp(h, agg, w1, b1, w2, b2):
    def body(h_ref, a_ref, w1_ref, b1_ref, w2_ref, b2_ref, o_ref):
        z = h_ref[...] + a_ref[...]
        y = jnp.dot(z, w1_ref[...], preferred_element_type=jnp.float32)
        y = jnp.maximum(y + b1_ref[...], 0.0)
        o = jnp.dot(y, w2_ref[...], preferred_element_type=jnp.float32)
        o_ref[...] = jnp.maximum(o + b2_ref[...], 0.0)

    return pl.pallas_call(
        body,
        grid=(_GRID,),
        in_specs=[
            pl.BlockSpec((_BLK, H), lambda i: (i, 0)),
            pl.BlockSpec((_BLK, H), lambda i: (i, 0)),
            pl.BlockSpec((H, H), lambda i: (0, 0)),
            pl.BlockSpec((1, H), lambda i: (0, 0)),
            pl.BlockSpec((H, H), lambda i: (0, 0)),
            pl.BlockSpec((1, H), lambda i: (0, 0)),
        ],
        out_specs=pl.BlockSpec((_BLK, H), lambda i: (i, 0)),
        out_shape=jax.ShapeDtypeStruct((N, H), jnp.float32),
    )(h, agg, w1, b1, w2, b2)


def _gather_rows(h, didx):
    @functools.partial(
        pl.kernel,
        out_type=jax.ShapeDtypeStruct((K, H), jnp.float32),
        mesh=_mesh(),
        compiler_params=pltpu.CompilerParams(use_tc_tiling_on_sc=False),
        scratch_types=[
            pltpu.VMEM((K,), jnp.int32),
            pltpu.VMEM((K, H), jnp.float32),
            pltpu.SemaphoreType.DMA,
        ],
    )
    def k(h_hbm, didx_hbm, out_hbm, didxv, dembv, sem):
        c = lax.axis_index("c")
        s = lax.axis_index("s")

        @pl.when((c == 0) & (s == 0))
        def _():
            pltpu.sync_copy(didx_hbm, didxv)
            pltpu.async_copy(h_hbm.at[didxv], dembv, sem).wait()
            pltpu.sync_copy(dembv, out_hbm)

    return k(h, didx)


def _heads(demb, pw, pb, vw, vb, nop):
    def body(d_ref, pw_ref, pb_ref, vw_ref, vb_ref, nop_ref, pol_ref, val_ref):
        d = d_ref[...]
        pol = jnp.dot(d, pw_ref[...], preferred_element_type=jnp.float32)
        pol_ref[...] = pol + pb_ref[...]
        m = (lax.broadcasted_iota(jnp.int32, (K, 1), 0)
             == nop_ref[0]).astype(jnp.float32)
        wait_row = jnp.sum(d * m, axis=0, keepdims=True)
        val = jnp.dot(wait_row, vw_ref[...],
                      preferred_element_type=jnp.float32)
        val_ref[...] = val + vb_ref[...]

    return pl.pallas_call(
        body,
        in_specs=[
            pl.BlockSpec(memory_space=pltpu.MemorySpace.VMEM),
            pl.BlockSpec(memory_space=pltpu.MemorySpace.VMEM),
            pl.BlockSpec(memory_space=pltpu.MemorySpace.VMEM),
            pl.BlockSpec(memory_space=pltpu.MemorySpace.VMEM),
            pl.BlockSpec(memory_space=pltpu.MemorySpace.VMEM),
            pl.BlockSpec(memory_space=pltpu.MemorySpace.SMEM),
        ],
        out_specs=[
            pl.BlockSpec(memory_space=pltpu.MemorySpace.VMEM),
            pl.BlockSpec(memory_space=pltpu.MemorySpace.VMEM),
        ],
        out_shape=[
            jax.ShapeDtypeStruct((K, 1), jnp.float32),
            jax.ShapeDtypeStruct((1, 1), jnp.float32),
        ],
    )(demb, pw, pb, vw, vb, nop)


def kernel(x, edge_index, defense_indices, nop_index,
           gin_w1_0, gin_b1_0, gin_w2_0, gin_b2_0,
           gin_w1_1, gin_b1_1, gin_w2_1, gin_b2_1,
           gin_w1_2, gin_b1_2, gin_w2_2, gin_b2_2,
           policy_w, policy_b, value_w, value_b):
    src = edge_index[0]
    dst = edge_index[1]
    pad_s = jnp.zeros((EPAD - E,), jnp.int32)
    pad_d = jnp.full((EPAD - E,), 1 << 30, jnp.int32)
    src2 = jnp.concatenate([src, pad_s]).reshape(EPAD // SUB, SUB)
    dst2 = jnp.concatenate([dst, pad_d]).reshape(EPAD // SUB, SUB)

    agg0 = _seg_sum_1d(x[:, 0], src2, dst2)
    h1 = _mlp0(x, agg0.reshape(N, 1),
               gin_w1_0, gin_b1_0.reshape(1, H), gin_w2_0, gin_b2_0.reshape(1, H))
    agg1 = _seg_sum_2d(h1, src2, dst2)
    h2 = _mlp(h1, agg1,
              gin_w1_1, gin_b1_1.reshape(1, H), gin_w2_1, gin_b2_1.reshape(1, H))
    agg2 = _seg_sum_2d(h2, src2, dst2)
    h3 = _mlp(h2, agg2,
              gin_w1_2, gin_b1_2.reshape(1, H), gin_w2_2, gin_b2_2.reshape(1, H))

    demb = _gather_rows(h3, defense_indices[0])
    nop = jnp.reshape(jnp.asarray(nop_index, jnp.int32), (1,))
    pol, val = _heads(demb, policy_w, policy_b.reshape(1, 1),
                      value_w, value_b.reshape(1, 1), nop)
    return pol[:, 0], val[0, 0]


# speedup vs baseline: 5.2315x; 1.1172x over previous
import functools

import jax
import jax.numpy as jnp
from jax import lax
from jax.experimental import pallas as pl
from jax.experimental.pallas import tpu as pltpu
from jax.experimental.pallas import tpu_sc as plsc

N = 50000
E = 800000
H = 64
K = 64

NC = 2
NS = 16
HALF = N // NC
ACC_ROWS = 25600
TRASH_MASK = 511
EPT = 51200
EPAD = EPT * NS
SUB = 128
CHUNKS_T = EPT // SUB
ZROWS = 1600
OUT_PT = 1560
OUT_PT1 = 1560

@functools.lru_cache(maxsize=None)
def _mesh():
    return plsc.VectorSubcoreMesh(core_axis_name="c", subcore_axis_name="s",
                                  num_cores=NC, num_subcores=NS)


def _seg_sum_2d(h, src1, dst1):
    @functools.partial(
        pl.kernel,
        out_type=jax.ShapeDtypeStruct((N, H), jnp.float32),
        mesh=_mesh(),
        compiler_params=pltpu.CompilerParams(use_tc_tiling_on_sc=False),
        scratch_types=[
            pltpu.VMEM((2 * SUB,), jnp.int32),
            pltpu.VMEM((2 * SUB,), jnp.int32),
            pltpu.VMEM((2, SUB), jnp.int32),
            pltpu.VMEM((2 * SUB, H), jnp.float32),
            pltpu.VMEM_SHARED((ACC_ROWS, H), jnp.float32),
            pltpu.SemaphoreType.DMA,
            pltpu.SemaphoreType.DMA,
            pltpu.SemaphoreType.DMA,
        ],
    )
    def k(h_hbm, src_hbm, dst_hbm, out_hbm, srcv, dstv, lidx, rows, acc,
          isem, gsem, ssem):
        c = lax.axis_index("c")
        s = lax.axis_index("s")
        zero16 = jnp.zeros((16,), jnp.float32)
        iota = lax.broadcasted_iota(jnp.int32, (16,), 0)

        def zrow(i, _):
            for kk in range(H // 16):
                rows[i, pl.ds(kk * 16, 16)] = zero16
            return 0

        lax.fori_loop(0, 128, zrow, 0)

        def zacc(i, _):
            pltpu.sync_copy(rows.at[pl.ds(0, 128), :],
                            acc.at[pl.ds(s * ZROWS + i * 128, 128), :])
            return 0

        lax.fori_loop(0, 12, zacc, 0)
        pltpu.sync_copy(rows.at[pl.ds(0, 64), :],
                        acc.at[pl.ds(s * ZROWS + 1536, 64), :])
        plsc.subcore_barrier()

        base_half = c * HALF
        e0 = s * EPT

        for b in range(2):
            pltpu.async_copy(src_hbm.at[pl.ds(e0 + b * SUB, SUB)],
                             srcv.at[pl.ds(b * SUB, SUB)], isem)
            pltpu.async_copy(dst_hbm.at[pl.ds(e0 + b * SUB, SUB)],
                             dstv.at[pl.ds(b * SUB, SUB)], isem)

        G = CHUNKS_T // 2

        def body(g, _):
            for b in range(2):
                i = 2 * g + b

                @pl.when(g > 0)
                def _():
                    pltpu.make_async_copy(
                        h_hbm.at[pl.ds(0, SUB), :],
                        rows.at[pl.ds(b * SUB, SUB), :], ssem).wait()

                pltpu.make_async_copy(src_hbm.at[pl.ds(0, SUB)],
                                      srcv.at[pl.ds(b * SUB, SUB)], isem).wait()
                pltpu.make_async_copy(dst_hbm.at[pl.ds(0, SUB)],
                                      dstv.at[pl.ds(b * SUB, SUB)], isem).wait()
                gcp = pltpu.async_copy(h_hbm.at[srcv.at[pl.ds(b * SUB, SUB)]],
                                       rows.at[pl.ds(b * SUB, SUB), :], gsem)
                for kk in range(SUB // 16):
                    d = dstv[pl.ds(b * SUB + kk * 16, 16)]
                    lo = d - base_half
                    ok = (lo >= 0) & (lo < HALF)
                    tr = HALF + ((iota + kk * 16 + s * 64 + i * 32)
                                 & TRASH_MASK)
                    lidx[b, pl.ds(kk * 16, 16)] = jnp.where(ok, lo, tr)
                gcp.wait()

                @pl.when(g < G - 1)
                def _():
                    pltpu.async_copy(src_hbm.at[pl.ds(e0 + (i + 2) * SUB, SUB)],
                                     srcv.at[pl.ds(b * SUB, SUB)], isem)
                    pltpu.async_copy(dst_hbm.at[pl.ds(e0 + (i + 2) * SUB, SUB)],
                                     dstv.at[pl.ds(b * SUB, SUB)], isem)

                pltpu.async_copy(rows.at[pl.ds(b * SUB, SUB), :],
                                 acc.at[lidx.at[b]], ssem, add=True)
            return 0

        lax.fori_loop(0, G, body, 0)
        for b in range(2):
            pltpu.make_async_copy(h_hbm.at[pl.ds(0, SUB), :],
                                  rows.at[pl.ds(b * SUB, SUB), :], ssem).wait()
        plsc.subcore_barrier()

        def cout(i, _):
            pltpu.sync_copy(acc.at[pl.ds(s * OUT_PT + i * 128, 128), :],
                            rows.at[pl.ds(0, 128), :])
            pltpu.sync_copy(
                rows.at[pl.ds(0, 128), :],
                out_hbm.at[pl.ds(c * HALF + s * OUT_PT + i * 128, 128), :])
            return 0

        lax.fori_loop(0, OUT_PT // 128, cout, 0)
        t0 = (OUT_PT // 128) * 128
        tl = OUT_PT - t0
        pltpu.sync_copy(acc.at[pl.ds(s * OUT_PT + t0, tl), :],
                        rows.at[pl.ds(0, tl), :])
        pltpu.sync_copy(rows.at[pl.ds(0, tl), :],
                        out_hbm.at[pl.ds(c * HALF + s * OUT_PT + t0, tl), :])

        @pl.when(s == 0)
        def _tail():
            rem = HALF - NS * OUT_PT
            pltpu.sync_copy(acc.at[pl.ds(NS * OUT_PT, rem), :],
                            rows.at[pl.ds(0, rem), :])
            pltpu.sync_copy(rows.at[pl.ds(0, rem), :],
                            out_hbm.at[pl.ds(c * HALF + NS * OUT_PT, rem), :])

    return k(h, src1, dst1)


def _seg_sum_1d(x1, src1, dst1):
    @functools.partial(
        pl.kernel,
        out_type=jax.ShapeDtypeStruct((N,), jnp.float32),
        mesh=_mesh(),
        compiler_params=pltpu.CompilerParams(use_tc_tiling_on_sc=False),
        scratch_types=[
            pltpu.VMEM((2 * SUB,), jnp.int32),
            pltpu.VMEM((2 * SUB,), jnp.int32),
            pltpu.VMEM((2, SUB), jnp.int32),
            pltpu.VMEM((2 * SUB,), jnp.float32),
            pltpu.VMEM((128,), jnp.float32),
            pltpu.VMEM_SHARED((ACC_ROWS,), jnp.float32),
            pltpu.SemaphoreType.DMA,
            pltpu.SemaphoreType.DMA,
            pltpu.SemaphoreType.DMA,
        ],
    )
    def k(x_hbm, src_hbm, dst_hbm, out_hbm, srcv, dstv, lidx, rows, zbuf, acc,
          isem, gsem, ssem):
        c = lax.axis_index("c")
        s = lax.axis_index("s")
        zero16 = jnp.zeros((16,), jnp.float32)
        iota = lax.broadcasted_iota(jnp.int32, (16,), 0)

        for kk in range(8):
            zbuf[pl.ds(kk * 16, 16)] = zero16

        def zacc(i, _):
            pltpu.sync_copy(zbuf, acc.at[pl.ds(s * ZROWS + i * 128, 128)])
            return 0

        lax.fori_loop(0, 12, zacc, 0)
        pltpu.sync_copy(zbuf.at[pl.ds(0, 64)],
                        acc.at[pl.ds(s * ZROWS + 1536, 64)])
        plsc.subcore_barrier()

        base_half = c * HALF
        e0 = s * EPT

        for b in range(2):
            pltpu.async_copy(src_hbm.at[pl.ds(e0 + b * SUB, SUB)],
                             srcv.at[pl.ds(b * SUB, SUB)], isem)
            pltpu.async_copy(dst_hbm.at[pl.ds(e0 + b * SUB, SUB)],
                             dstv.at[pl.ds(b * SUB, SUB)], isem)

        G = CHUNKS_T // 2

        def body(g, _):
            for b in range(2):
                i = 2 * g + b

                @pl.when(g > 0)
                def _():
                    pltpu.make_async_copy(
                        x_hbm.at[pl.ds(0, SUB)],
                        rows.at[pl.ds(b * SUB, SUB)], ssem).wait()

                pltpu.make_async_copy(src_hbm.at[pl.ds(0, SUB)],
                                      srcv.at[pl.ds(b * SUB, SUB)], isem).wait()
                pltpu.make_async_copy(dst_hbm.at[pl.ds(0, SUB)],
                                      dstv.at[pl.ds(b * SUB, SUB)], isem).wait()
                gcp = pltpu.async_copy(x_hbm.at[srcv.at[pl.ds(b * SUB, SUB)]],
                                       rows.at[pl.ds(b * SUB, SUB)], gsem)
                for kk in range(SUB // 16):
                    d = dstv[pl.ds(b * SUB + kk * 16, 16)]
                    lo = d - base_half
                    ok = (lo >= 0) & (lo < HALF)
                    tr = HALF + ((iota + kk * 16 + s * 64 + i * 32)
                                 & TRASH_MASK)
                    lidx[b, pl.ds(kk * 16, 16)] = jnp.where(ok, lo, tr)
                gcp.wait()

                @pl.when(g < G - 1)
                def _():
                    pltpu.async_copy(src_hbm.at[pl.ds(e0 + (i + 2) * SUB, SUB)],
                                     srcv.at[pl.ds(b * SUB, SUB)], isem)
                    pltpu.async_copy(dst_hbm.at[pl.ds(e0 + (i + 2) * SUB, SUB)],
                                     dstv.at[pl.ds(b * SUB, SUB)], isem)

                pltpu.async_copy(rows.at[pl.ds(b * SUB, SUB)],
                                 acc.at[lidx.at[b]], ssem, add=True)
            return 0

        lax.fori_loop(0, G, body, 0)
        for b in range(2):
            pltpu.make_async_copy(x_hbm.at[pl.ds(0, SUB)],
                                  rows.at[pl.ds(b * SUB, SUB)], ssem).wait()
        plsc.subcore_barrier()

        def cout(i, _):
            pltpu.sync_copy(acc.at[pl.ds(s * OUT_PT1 + i * 128, 128)], zbuf)
            pltpu.sync_copy(
                zbuf, out_hbm.at[pl.ds(c * HALF + s * OUT_PT1 + i * 128, 128)])
            return 0

        lax.fori_loop(0, OUT_PT1 // 128, cout, 0)
        t0 = (OUT_PT1 // 128) * 128
        tl = OUT_PT1 - t0
        pltpu.sync_copy(acc.at[pl.ds(s * OUT_PT1 + t0, tl)],
                        zbuf.at[pl.ds(0, tl)])
        pltpu.sync_copy(zbuf.at[pl.ds(0, tl)],
                        out_hbm.at[pl.ds(c * HALF + s * OUT_PT1 + t0, tl)])

        @pl.when(s == 0)
        def _tail():
            rem = HALF - NS * OUT_PT1
            pltpu.sync_copy(acc.at[pl.ds(NS * OUT_PT1, rem)],
                            zbuf.at[pl.ds(0, rem)])
            pltpu.sync_copy(zbuf.at[pl.ds(0, rem)],
                            out_hbm.at[pl.ds(c * HALF + NS * OUT_PT1, rem)])

    return k(x1, src1, dst1)


_BLK = 2000
_GRID = N // _BLK


def _mlp0(x, agg0, w1, b1, w2, b2):
    def body(x_ref, a_ref, w1_ref, b1_ref, w2_ref, b2_ref, o_ref):
        z = x_ref[...] + a_ref[...]
        y = jnp.maximum(z * w1_ref[...] + b1_ref[...], 0.0)
        h = jnp.dot(y, w2_ref[...], preferred_element_type=jnp.float32)
        o_ref[...] = jnp.maximum(h + b2_ref[...], 0.0)

    return pl.pallas_call(
        body,
        grid=(_GRID,),
        in_specs=[
            pl.BlockSpec((_BLK, 1), lambda i: (i, 0)),
            pl.BlockSpec((_BLK, 1), lambda i: (i, 0)),
            pl.BlockSpec((1, H), lambda i: (0, 0)),
            pl.BlockSpec((1, H), lambda i: (0, 0)),
            pl.BlockSpec((H, H), lambda i: (0, 0)),
            pl.BlockSpec((1, H), lambda i: (0, 0)),
        ],
        out_specs=pl.BlockSpec((_BLK, H), lambda i: (i, 0)),
        out_shape=jax.ShapeDtypeStruct((N, H), jnp.float32),
    )(x, agg0, w1, b1, w2, b2)


def _mlp(h, agg, w1, b1, w2, b2):
    def body(h_ref, a_ref, w1_ref, b1_ref, w2_ref, b2_ref, o_ref):
        z = h_ref[...] + a_ref[...]
        y = jnp.dot(z, w1_ref[...], preferred_element_type=jnp.float32)
        y = jnp.maximum(y + b1_ref[...], 0.0)
        o = jnp.dot(y, w2_ref[...], preferred_element_type=jnp.float32)
        o_ref[...] = jnp.maximum(o + b2_ref[...], 0.0)

    return pl.pallas_call(
        body,
        grid=(_GRID,),
        in_specs=[
            pl.BlockSpec((_BLK, H), lambda i: (i, 0)),
            pl.BlockSpec((_BLK, H), lambda i: (i, 0)),
            pl.BlockSpec((H, H), lambda i: (0, 0)),
            pl.BlockSpec((1, H), lambda i: (0, 0)),
            pl.BlockSpec((H, H), lambda i: (0, 0)),
            pl.BlockSpec((1, H), lambda i: (0, 0)),
        ],
        out_specs=pl.BlockSpec((_BLK, H), lambda i: (i, 0)),
        out_shape=jax.ShapeDtypeStruct((N, H), jnp.float32),
    )(h, agg, w1, b1, w2, b2)


def _gather_rows(h, didx):
    @functools.partial(
        pl.kernel,
        out_type=jax.ShapeDtypeStruct((K, H), jnp.float32),
        mesh=_mesh(),
        compiler_params=pltpu.CompilerParams(use_tc_tiling_on_sc=False),
        scratch_types=[
            pltpu.VMEM((K,), jnp.int32),
            pltpu.VMEM((K, H), jnp.float32),
            pltpu.SemaphoreType.DMA,
        ],
    )
    def k(h_hbm, didx_hbm, out_hbm, didxv, dembv, sem):
        c = lax.axis_index("c")
        s = lax.axis_index("s")

        @pl.when((c == 0) & (s == 0))
        def _():
            pltpu.sync_copy(didx_hbm, didxv)
            pltpu.async_copy(h_hbm.at[didxv], dembv, sem).wait()
            pltpu.sync_copy(dembv, out_hbm)

    return k(h, didx)


def _heads(demb, pw, pb, vw, vb, nop):
    def body(d_ref, pw_ref, pb_ref, vw_ref, vb_ref, nop_ref, pol_ref, val_ref):
        d = d_ref[...]
        pol = jnp.dot(d, pw_ref[...], preferred_element_type=jnp.float32)
        pol_ref[...] = pol + pb_ref[...]
        m = (lax.broadcasted_iota(jnp.int32, (K, 1), 0)
             == nop_ref[0]).astype(jnp.float32)
        wait_row = jnp.sum(d * m, axis=0, keepdims=True)
        val = jnp.dot(wait_row, vw_ref[...],
                      preferred_element_type=jnp.float32)
        val_ref[...] = val + vb_ref[...]

    return pl.pallas_call(
        body,
        in_specs=[
            pl.BlockSpec(memory_space=pltpu.MemorySpace.VMEM),
            pl.BlockSpec(memory_space=pltpu.MemorySpace.VMEM),
            pl.BlockSpec(memory_space=pltpu.MemorySpace.VMEM),
            pl.BlockSpec(memory_space=pltpu.MemorySpace.VMEM),
            pl.BlockSpec(memory_space=pltpu.MemorySpace.VMEM),
            pl.BlockSpec(memory_space=pltpu.MemorySpace.SMEM),
        ],
        out_specs=[
            pl.BlockSpec(memory_space=pltpu.MemorySpace.VMEM),
            pl.BlockSpec(memory_space=pltpu.MemorySpace.VMEM),
        ],
        out_shape=[
            jax.ShapeDtypeStruct((K, 1), jnp.float32),
            jax.ShapeDtypeStruct((1, 1), jnp.float32),
        ],
    )(demb, pw, pb, vw, vb, nop)


def kernel(x, edge_index, defense_indices, nop_index,
           gin_w1_0, gin_b1_0, gin_w2_0, gin_b2_0,
           gin_w1_1, gin_b1_1, gin_w2_1, gin_b2_1,
           gin_w1_2, gin_b1_2, gin_w2_2, gin_b2_2,
           policy_w, policy_b, value_w, value_b):
    src = edge_index[0]
    dst = edge_index[1]
    pad_s = jnp.zeros((EPAD - E,), jnp.int32)
    pad_d = jnp.full((EPAD - E,), 1 << 30, jnp.int32)
    src1 = jnp.concatenate([src, pad_s])
    dst1 = jnp.concatenate([dst, pad_d])

    agg0 = _seg_sum_1d(x[:, 0], src1, dst1)
    h1 = _mlp0(x, agg0.reshape(N, 1),
               gin_w1_0, gin_b1_0.reshape(1, H), gin_w2_0, gin_b2_0.reshape(1, H))
    agg1 = _seg_sum_2d(h1, src1, dst1)
    h2 = _mlp(h1, agg1,
              gin_w1_1, gin_b1_1.reshape(1, H), gin_w2_1, gin_b2_1.reshape(1, H))
    agg2 = _seg_sum_2d(h2, src1, dst1)
    h3 = _mlp(h2, agg2,
              gin_w1_2, gin_b1_2.reshape(1, H), gin_w2_2, gin_b2_2.reshape(1, H))

    demb = _gather_rows(h3, defense_indices[0])
    nop = jnp.reshape(jnp.asarray(nop_index, jnp.int32), (1,))
    pol, val = _heads(demb, policy_w, policy_b.reshape(1, 1),
                      value_w, value_b.reshape(1, 1), nop)
    return pol[:, 0], val[0, 0]


# speedup vs baseline: 12.6064x; 2.4097x over previous
import functools

import jax
import jax.numpy as jnp
from jax import lax
from jax.experimental import pallas as pl
from jax.experimental.pallas import tpu as pltpu
from jax.experimental.pallas import tpu_sc as plsc

N = 50000
E = 800000
H = 64
K = 64

NC = 2
NS = 16
HALF = N // NC
ACC_ROWS = 25600
TRASH_MASK = 511
EPT = 51200
EPAD = EPT * NS
SUB = 128
ZROWS = 1600
OUT_PT = 1560
OUT_PT1 = 1560

CAP = 13312
PREP_EPT = EPAD // 32
HREG = 32 * CAP
EPT_SEG = 2 * CAP
G_SEG = EPT_SEG // SUB // 2

@functools.lru_cache(maxsize=None)
def _mesh():
    return plsc.VectorSubcoreMesh(core_axis_name="c", subcore_axis_name="s",
                                  num_cores=NC, num_subcores=NS)


def _partition(src1, dst1):
    @functools.partial(
        pl.kernel,
        out_type=(jax.ShapeDtypeStruct((2 * HREG,), jnp.int32),
                  jax.ShapeDtypeStruct((2 * HREG,), jnp.int32)),
        mesh=_mesh(),
        compiler_params=pltpu.CompilerParams(use_tc_tiling_on_sc=False,
                                             needs_layout_passes=False),
        scratch_types=[
            pltpu.VMEM((1024,), jnp.int32),
            pltpu.VMEM((1024,), jnp.int32),
            pltpu.VMEM((CAP,), jnp.int32),
            pltpu.VMEM((CAP,), jnp.int32),
            pltpu.VMEM((CAP,), jnp.int32),
            pltpu.VMEM((CAP,), jnp.int32),
        ],
    )
    def k(src_hbm, dst_hbm, outs_hbm, outd_hbm, sv, dv,
          s0b, d0b, s1b, d1b):
        c = lax.axis_index("c")
        s = lax.axis_index("s")
        w = s * NC + c
        iota = lax.broadcasted_iota(jnp.int32, (16,), 0)
        sent16 = jnp.full((16,), 1 << 30, jnp.int32)

        def fill(i, _):
            base = i * 16 + iota
            dummy_src = base & 127
            plsc.store_scatter(s0b, [base], dummy_src)
            plsc.store_scatter(d0b, [base], sent16)
            plsc.store_scatter(s1b, [base], dummy_src)
            plsc.store_scatter(d1b, [base], sent16)
            return 0

        lax.fori_loop(0, CAP // 16, fill, 0)

        e0 = w * PREP_EPT

        def chunk(i, carry):
            off0, off1 = carry
            pltpu.sync_copy(src_hbm.at[pl.ds(e0 + i * 1024, 1024)], sv)
            pltpu.sync_copy(dst_hbm.at[pl.ds(e0 + i * 1024, 1024)], dv)
            for t in range(64):
                s16 = sv[pl.ds(t * 16, 16)]
                d16 = dv[pl.ds(t * 16, 16)]
                m0 = d16 < HALF
                m1 = (d16 >= HALF) & (d16 < N)
                m0i = m0.astype(jnp.int32)
                m1i = m1.astype(jnp.int32)
                pos0 = jnp.minimum(off0 + plsc.cumsum(m0i) - m0i, CAP - 1)
                pos1 = jnp.minimum(off1 + plsc.cumsum(m1i) - m1i, CAP - 1)
                plsc.store_scatter(s0b, [pos0], s16, mask=m0)
                plsc.store_scatter(d0b, [pos0], d16, mask=m0)
                plsc.store_scatter(s1b, [pos1], s16, mask=m1)
                plsc.store_scatter(d1b, [pos1], d16 - HALF, mask=m1)
                off0 = off0 + jnp.sum(m0i)
                off1 = off1 + jnp.sum(m1i)
            return off0, off1

        lax.fori_loop(0, PREP_EPT // 1024, chunk, (jnp.int32(0), jnp.int32(0)))

        pltpu.sync_copy(s0b, outs_hbm.at[pl.ds(w * CAP, CAP)])
        pltpu.sync_copy(d0b, outd_hbm.at[pl.ds(w * CAP, CAP)])
        pltpu.sync_copy(s1b, outs_hbm.at[pl.ds(HREG + w * CAP, CAP)])
        pltpu.sync_copy(d1b, outd_hbm.at[pl.ds(HREG + w * CAP, CAP)])

    return k(src1, dst1)


def _seg_sum_2d(h, src1, dst1):
    @functools.partial(
        pl.kernel,
        out_type=jax.ShapeDtypeStruct((N, H), jnp.float32),
        mesh=_mesh(),
        compiler_params=pltpu.CompilerParams(use_tc_tiling_on_sc=False),
        scratch_types=[
            pltpu.VMEM((2 * SUB,), jnp.int32),
            pltpu.VMEM((2 * SUB,), jnp.int32),
            pltpu.VMEM((2, SUB), jnp.int32),
            pltpu.VMEM((2 * SUB, H), jnp.float32),
            pltpu.VMEM_SHARED((ACC_ROWS, H), jnp.float32),
            pltpu.SemaphoreType.DMA,
            pltpu.SemaphoreType.DMA,
            pltpu.SemaphoreType.DMA,
        ],
    )
    def k(h_hbm, src_hbm, dst_hbm, out_hbm, srcv, dstv, lidx, rows, acc,
          isem, gsem, ssem):
        c = lax.axis_index("c")
        s = lax.axis_index("s")
        zero16 = jnp.zeros((16,), jnp.float32)
        iota = lax.broadcasted_iota(jnp.int32, (16,), 0)

        def zrow(i, _):
            for kk in range(H // 16):
                rows[i, pl.ds(kk * 16, 16)] = zero16
            return 0

        lax.fori_loop(0, 128, zrow, 0)

        def zacc(i, _):
            pltpu.sync_copy(rows.at[pl.ds(0, 128), :],
                            acc.at[pl.ds(s * ZROWS + i * 128, 128), :])
            return 0

        lax.fori_loop(0, 12, zacc, 0)
        pltpu.sync_copy(rows.at[pl.ds(0, 64), :],
                        acc.at[pl.ds(s * ZROWS + 1536, 64), :])
        plsc.subcore_barrier()

        e0 = c * HREG + s * EPT_SEG

        for b in range(2):
            pltpu.async_copy(src_hbm.at[pl.ds(e0 + b * SUB, SUB)],
                             srcv.at[pl.ds(b * SUB, SUB)], isem)
            pltpu.async_copy(dst_hbm.at[pl.ds(e0 + b * SUB, SUB)],
                             dstv.at[pl.ds(b * SUB, SUB)], isem)

        G = G_SEG

        def body(g, _):
            for b in range(2):
                i = 2 * g + b

                @pl.when(g > 0)
                def _():
                    pltpu.make_async_copy(
                        h_hbm.at[pl.ds(0, SUB), :],
                        rows.at[pl.ds(b * SUB, SUB), :], ssem).wait()

                pltpu.make_async_copy(src_hbm.at[pl.ds(0, SUB)],
                                      srcv.at[pl.ds(b * SUB, SUB)], isem).wait()
                pltpu.make_async_copy(dst_hbm.at[pl.ds(0, SUB)],
                                      dstv.at[pl.ds(b * SUB, SUB)], isem).wait()
                gcp = pltpu.async_copy(h_hbm.at[srcv.at[pl.ds(b * SUB, SUB)]],
                                       rows.at[pl.ds(b * SUB, SUB), :], gsem)
                for kk in range(SUB // 16):
                    d = dstv[pl.ds(b * SUB + kk * 16, 16)]
                    ok = d < HALF
                    tr = HALF + ((iota + kk * 16 + s * 64 + i * 32)
                                 & TRASH_MASK)
                    lidx[b, pl.ds(kk * 16, 16)] = jnp.where(ok, d, tr)
                gcp.wait()

                @pl.when(g < G - 1)
                def _():
                    pltpu.async_copy(src_hbm.at[pl.ds(e0 + (i + 2) * SUB, SUB)],
                                     srcv.at[pl.ds(b * SUB, SUB)], isem)
                    pltpu.async_copy(dst_hbm.at[pl.ds(e0 + (i + 2) * SUB, SUB)],
                                     dstv.at[pl.ds(b * SUB, SUB)], isem)

                pltpu.async_copy(rows.at[pl.ds(b * SUB, SUB), :],
                                 acc.at[lidx.at[b]], ssem, add=True)
            return 0

        lax.fori_loop(0, G, body, 0)
        for b in range(2):
            pltpu.make_async_copy(h_hbm.at[pl.ds(0, SUB), :],
                                  rows.at[pl.ds(b * SUB, SUB), :], ssem).wait()
        plsc.subcore_barrier()

        def cout(i, _):
            pltpu.sync_copy(acc.at[pl.ds(s * OUT_PT + i * 128, 128), :],
                            rows.at[pl.ds(0, 128), :])
            pltpu.sync_copy(
                rows.at[pl.ds(0, 128), :],
                out_hbm.at[pl.ds(c * HALF + s * OUT_PT + i * 128, 128), :])
            return 0

        lax.fori_loop(0, OUT_PT // 128, cout, 0)
        t0 = (OUT_PT // 128) * 128
        tl = OUT_PT - t0
        pltpu.sync_copy(acc.at[pl.ds(s * OUT_PT + t0, tl), :],
                        rows.at[pl.ds(0, tl), :])
        pltpu.sync_copy(rows.at[pl.ds(0, tl), :],
                        out_hbm.at[pl.ds(c * HALF + s * OUT_PT + t0, tl), :])

        @pl.when(s == 0)
        def _tail():
            rem = HALF - NS * OUT_PT
            pltpu.sync_copy(acc.at[pl.ds(NS * OUT_PT, rem), :],
                            rows.at[pl.ds(0, rem), :])
            pltpu.sync_copy(rows.at[pl.ds(0, rem), :],
                            out_hbm.at[pl.ds(c * HALF + NS * OUT_PT, rem), :])

    return k(h, src1, dst1)


def _seg_sum_1d(x1, src1, dst1):
    @functools.partial(
        pl.kernel,
        out_type=jax.ShapeDtypeStruct((N,), jnp.float32),
        mesh=_mesh(),
        compiler_params=pltpu.CompilerParams(use_tc_tiling_on_sc=False),
        scratch_types=[
            pltpu.VMEM((2 * SUB,), jnp.int32),
            pltpu.VMEM((2 * SUB,), jnp.int32),
            pltpu.VMEM((2, SUB), jnp.int32),
            pltpu.VMEM((2 * SUB,), jnp.float32),
            pltpu.VMEM((128,), jnp.float32),
            pltpu.VMEM_SHARED((ACC_ROWS,), jnp.float32),
            pltpu.SemaphoreType.DMA,
            pltpu.SemaphoreType.DMA,
            pltpu.SemaphoreType.DMA,
        ],
    )
    def k(x_hbm, src_hbm, dst_hbm, out_hbm, srcv, dstv, lidx, rows, zbuf, acc,
          isem, gsem, ssem):
        c = lax.axis_index("c")
        s = lax.axis_index("s")
        zero16 = jnp.zeros((16,), jnp.float32)
        iota = lax.broadcasted_iota(jnp.int32, (16,), 0)

        for kk in range(8):
            zbuf[pl.ds(kk * 16, 16)] = zero16

        def zacc(i, _):
            pltpu.sync_copy(zbuf, acc.at[pl.ds(s * ZROWS + i * 128, 128)])
            return 0

        lax.fori_loop(0, 12, zacc, 0)
        pltpu.sync_copy(zbuf.at[pl.ds(0, 64)],
                        acc.at[pl.ds(s * ZROWS + 1536, 64)])
        plsc.subcore_barrier()

        e0 = c * HREG + s * EPT_SEG

        for b in range(2):
            pltpu.async_copy(src_hbm.at[pl.ds(e0 + b * SUB, SUB)],
                             srcv.at[pl.ds(b * SUB, SUB)], isem)
            pltpu.async_copy(dst_hbm.at[pl.ds(e0 + b * SUB, SUB)],
                             dstv.at[pl.ds(b * SUB, SUB)], isem)

        G = G_SEG

        def body(g, _):
            for b in range(2):
                i = 2 * g + b

                @pl.when(g > 0)
                def _():
                    pltpu.make_async_copy(
                        x_hbm.at[pl.ds(0, SUB)],
                        rows.at[pl.ds(b * SUB, SUB)], ssem).wait()

                pltpu.make_async_copy(src_hbm.at[pl.ds(0, SUB)],
                                      srcv.at[pl.ds(b * SUB, SUB)], isem).wait()
                pltpu.make_async_copy(dst_hbm.at[pl.ds(0, SUB)],
                                      dstv.at[pl.ds(b * SUB, SUB)], isem).wait()
                gcp = pltpu.async_copy(x_hbm.at[srcv.at[pl.ds(b * SUB, SUB)]],
                                       rows.at[pl.ds(b * SUB, SUB)], gsem)
                for kk in range(SUB // 16):
                    d = dstv[pl.ds(b * SUB + kk * 16, 16)]
                    ok = d < HALF
                    tr = HALF + ((iota + kk * 16 + s * 64 + i * 32)
                                 & TRASH_MASK)
                    lidx[b, pl.ds(kk * 16, 16)] = jnp.where(ok, d, tr)
                gcp.wait()

                @pl.when(g < G - 1)
                def _():
                    pltpu.async_copy(src_hbm.at[pl.ds(e0 + (i + 2) * SUB, SUB)],
                                     srcv.at[pl.ds(b * SUB, SUB)], isem)
                    pltpu.async_copy(dst_hbm.at[pl.ds(e0 + (i + 2) * SUB, SUB)],
                                     dstv.at[pl.ds(b * SUB, SUB)], isem)

                pltpu.async_copy(rows.at[pl.ds(b * SUB, SUB)],
                                 acc.at[lidx.at[b]], ssem, add=True)
            return 0

        lax.fori_loop(0, G, body, 0)
        for b in range(2):
            pltpu.make_async_copy(x_hbm.at[pl.ds(0, SUB)],
                                  rows.at[pl.ds(b * SUB, SUB)], ssem).wait()
        plsc.subcore_barrier()

        def cout(i, _):
            pltpu.sync_copy(acc.at[pl.ds(s * OUT_PT1 + i * 128, 128)], zbuf)
            pltpu.sync_copy(
                zbuf, out_hbm.at[pl.ds(c * HALF + s * OUT_PT1 + i * 128, 128)])
            return 0

        lax.fori_loop(0, OUT_PT1 // 128, cout, 0)
        t0 = (OUT_PT1 // 128) * 128
        tl = OUT_PT1 - t0
        pltpu.sync_copy(acc.at[pl.ds(s * OUT_PT1 + t0, tl)],
                        zbuf.at[pl.ds(0, tl)])
        pltpu.sync_copy(zbuf.at[pl.ds(0, tl)],
                        out_hbm.at[pl.ds(c * HALF + s * OUT_PT1 + t0, tl)])

        @pl.when(s == 0)
        def _tail():
            rem = HALF - NS * OUT_PT1
            pltpu.sync_copy(acc.at[pl.ds(NS * OUT_PT1, rem)],
                            zbuf.at[pl.ds(0, rem)])
            pltpu.sync_copy(zbuf.at[pl.ds(0, rem)],
                            out_hbm.at[pl.ds(c * HALF + NS * OUT_PT1, rem)])

    return k(x1, src1, dst1)


_BLK = 2000
_GRID = N // _BLK


def _mlp0(x, agg0, w1, b1, w2, b2):
    def body(x_ref, a_ref, w1_ref, b1_ref, w2_ref, b2_ref, o_ref):
        z = x_ref[...] + a_ref[...]
        y = jnp.maximum(z * w1_ref[...] + b1_ref[...], 0.0)
        h = jnp.dot(y, w2_ref[...], preferred_element_type=jnp.float32)
        o_ref[...] = jnp.maximum(h + b2_ref[...], 0.0)

    return pl.pallas_call(
        body,
        grid=(_GRID,),
        in_specs=[
            pl.BlockSpec((_BLK, 1), lambda i: (i, 0)),
            pl.BlockSpec((_BLK, 1), lambda i: (i, 0)),
            pl.BlockSpec((1, H), lambda i: (0, 0)),
            pl.BlockSpec((1, H), lambda i: (0, 0)),
            pl.BlockSpec((H, H), lambda i: (0, 0)),
            pl.BlockSpec((1, H), lambda i: (0, 0)),
        ],
        out_specs=pl.BlockSpec((_BLK, H), lambda i: (i, 0)),
        out_shape=jax.ShapeDtypeStruct((N, H), jnp.float32),
    )(x, agg0, w1, b1, w2, b2)


def _mlp(h, agg, w1, b1, w2, b2):
    def body(h_ref, a_ref, w1_ref, b1_ref, w2_ref, b2_ref, o_ref):
        z = h_ref[...] + a_ref[...]
        y = jnp.dot(z, w1_ref[...], preferred_element_type=jnp.float32)
        y = jnp.maximum(y + b1_ref[...], 0.0)
        o = jnp.dot(y, w2_ref[...], preferred_element_type=jnp.float32)
        o_ref[...] = jnp.maximum(o + b2_ref[...], 0.0)

    return pl.pallas_call(
        body,
        grid=(_GRID,),
        in_specs=[
            pl.BlockSpec((_BLK, H), lambda i: (i, 0)),
            pl.BlockSpec((_BLK, H), lambda i: (i, 0)),
            pl.BlockSpec((H, H), lambda i: (0, 0)),
            pl.BlockSpec((1, H), lambda i: (0, 0)),
            pl.BlockSpec((H, H), lambda i: (0, 0)),
            pl.BlockSpec((1, H), lambda i: (0, 0)),
        ],
        out_specs=pl.BlockSpec((_BLK, H), lambda i: (i, 0)),
        out_shape=jax.ShapeDtypeStruct((N, H), jnp.float32),
    )(h, agg, w1, b1, w2, b2)


def _gather_rows(h, didx):
    @functools.partial(
        pl.kernel,
        out_type=jax.ShapeDtypeStruct((K, H), jnp.float32),
        mesh=_mesh(),
        compiler_params=pltpu.CompilerParams(use_tc_tiling_on_sc=False),
        scratch_types=[
            pltpu.VMEM((K,), jnp.int32),
            pltpu.VMEM((K, H), jnp.float32),
            pltpu.SemaphoreType.DMA,
        ],
    )
    def k(h_hbm, didx_hbm, out_hbm, didxv, dembv, sem):
        c = lax.axis_index("c")
        s = lax.axis_index("s")

        @pl.when((c == 0) & (s == 0))
        def _():
            pltpu.sync_copy(didx_hbm, didxv)
            pltpu.async_copy(h_hbm.at[didxv], dembv, sem).wait()
            pltpu.sync_copy(dembv, out_hbm)

    return k(h, didx)


def _heads(demb, pw, pb, vw, vb, nop):
    def body(d_ref, pw_ref, pb_ref, vw_ref, vb_ref, nop_ref, pol_ref, val_ref):
        d = d_ref[...]
        pol = jnp.dot(d, pw_ref[...], preferred_element_type=jnp.float32)
        pol_ref[...] = pol + pb_ref[...]
        m = (lax.broadcasted_iota(jnp.int32, (K, 1), 0)
             == nop_ref[0]).astype(jnp.float32)
        wait_row = jnp.sum(d * m, axis=0, keepdims=True)
        val = jnp.dot(wait_row, vw_ref[...],
                      preferred_element_type=jnp.float32)
        val_ref[...] = val + vb_ref[...]

    return pl.pallas_call(
        body,
        in_specs=[
            pl.BlockSpec(memory_space=pltpu.MemorySpace.VMEM),
            pl.BlockSpec(memory_space=pltpu.MemorySpace.VMEM),
            pl.BlockSpec(memory_space=pltpu.MemorySpace.VMEM),
            pl.BlockSpec(memory_space=pltpu.MemorySpace.VMEM),
            pl.BlockSpec(memory_space=pltpu.MemorySpace.VMEM),
            pl.BlockSpec(memory_space=pltpu.MemorySpace.SMEM),
        ],
        out_specs=[
            pl.BlockSpec(memory_space=pltpu.MemorySpace.VMEM),
            pl.BlockSpec(memory_space=pltpu.MemorySpace.VMEM),
        ],
        out_shape=[
            jax.ShapeDtypeStruct((K, 1), jnp.float32),
            jax.ShapeDtypeStruct((1, 1), jnp.float32),
        ],
    )(demb, pw, pb, vw, vb, nop)


def kernel(x, edge_index, defense_indices, nop_index,
           gin_w1_0, gin_b1_0, gin_w2_0, gin_b2_0,
           gin_w1_1, gin_b1_1, gin_w2_1, gin_b2_1,
           gin_w1_2, gin_b1_2, gin_w2_2, gin_b2_2,
           policy_w, policy_b, value_w, value_b):
    src = edge_index[0]
    dst = edge_index[1]
    pad_s = jnp.zeros((EPAD - E,), jnp.int32)
    pad_d = jnp.full((EPAD - E,), 1 << 30, jnp.int32)
    src1 = jnp.concatenate([src, pad_s])
    dst1 = jnp.concatenate([dst, pad_d])

    ps, pd = _partition(src1, dst1)
    agg0 = _seg_sum_1d(x[:, 0], ps, pd)
    h1 = _mlp0(x, agg0.reshape(N, 1),
               gin_w1_0, gin_b1_0.reshape(1, H), gin_w2_0, gin_b2_0.reshape(1, H))
    agg1 = _seg_sum_2d(h1, ps, pd)
    h2 = _mlp(h1, agg1,
              gin_w1_1, gin_b1_1.reshape(1, H), gin_w2_1, gin_b2_1.reshape(1, H))
    agg2 = _seg_sum_2d(h2, ps, pd)
    h3 = _mlp(h2, agg2,
              gin_w1_2, gin_b1_2.reshape(1, H), gin_w2_2, gin_b2_2.reshape(1, H))

    demb = _gather_rows(h3, defense_indices[0])
    nop = jnp.reshape(jnp.asarray(nop_index, jnp.int32), (1,))
    pol, val = _heads(demb, policy_w, policy_b.reshape(1, 1),
                      value_w, value_b.reshape(1, 1), nop)
    return pol[:, 0], val[0, 0]
